# baseline (speedup 1.0000x reference)
import functools

import jax
import jax.numpy as jnp
from jax import lax
from jax.experimental import pallas as pl
from jax.experimental.pallas import tpu as pltpu
from jax.experimental.pallas import tpu_sc as plsc

N = 10000
E = 640000
F_IN = 128
H = 32
C = 7
CPAD = 16

NC = 2
NS = 16
NW = NC * NS
L = 16

CH = 128
NROWS = E // CH
RW = 156
RX = NROWS - NW * RW
RMAX = RW + RX
NB = 4
RPAIR = RW // (2 * NB)
RPT = N // NS
NACC = N

_mesh = plsc.VectorSubcoreMesh(core_axis_name="c", subcore_axis_name="s")


@functools.partial(
    pl.kernel,
    out_type=jax.ShapeDtypeStruct((NW, N), jnp.float32),
    mesh=_mesh,
    compiler_params=pltpu.CompilerParams(needs_layout_passes=False, use_tc_tiling_on_sc=False),
    scratch_types=[
        pltpu.VMEM((RMAX, CH), jnp.int32),
        pltpu.VMEM((N,), jnp.float32),
    ],
)
def _deg_kernel(e3_hbm, out_hbm, dst_v, hist_v):
    wid = lax.axis_index("c") * NS + lax.axis_index("s")
    last = wid == NW - 1
    rbase = wid * RW
    pltpu.sync_copy(e3_hbm.at[1, pl.ds(rbase, RW)], dst_v.at[pl.ds(0, RW)])

    @pl.when(last)
    def _():
        pltpu.sync_copy(e3_hbm.at[1, pl.ds(rbase + RW, RX)],
                        dst_v.at[pl.ds(RW, RX)])

    zeros = jnp.zeros((L,), jnp.float32)

    def zbody(i, _):
        hist_v[pl.ds(i * L, L)] = zeros
        return 0

    lax.fori_loop(0, N // L, zbody, 0)

    ones = jnp.ones((L,), jnp.float32)
    nrows = jnp.where(last, RMAX, RW)

    def body(i, _):
        for k in range(CH // L):
            plsc.addupdate_scatter(hist_v, [dst_v[i, pl.ds(k * L, L)]], ones)
        return 0

    lax.fori_loop(0, nrows, body, 0)
    pltpu.sync_copy(hist_v, out_hbm.at[wid])


def _make_msg_kernel(feat):
    @functools.partial(
        pl.kernel,
        out_type=jax.ShapeDtypeStruct((NC, N, feat), jnp.float32),
        mesh=_mesh,
        compiler_params=pltpu.CompilerParams(needs_layout_passes=False, use_tc_tiling_on_sc=False),
        scratch_types=[
            pltpu.VMEM((RMAX, CH), jnp.int32),
            pltpu.VMEM((RMAX, CH), jnp.int32),
            pltpu.VMEM((2 * NB, CH, feat), jnp.float32),
            pltpu.VMEM((RPT // 5, feat), jnp.float32),
            pltpu.VMEM_SHARED((NACC, feat), jnp.float32),
            pltpu.VMEM_SHARED((N, feat), jnp.float32),
            pltpu.SemaphoreType.DMA,
            pltpu.SemaphoreType.DMA,
            pltpu.SemaphoreType.DMA,
            pltpu.SemaphoreType.DMA,
        ],
    )
    def _msg_kernel(g_hbm, e3_hbm, out_hbm,
                    src_v, dst_v, rows_v, zbuf_v, acc_sh, gsrc_sh,
                    gsa, gsb, ssa, ssb):
        c = lax.axis_index("c")
        s = lax.axis_index("s")
        wid = c * NS + s
        last = wid == NW - 1
        rbase = wid * RW

        pltpu.sync_copy(e3_hbm.at[0, pl.ds(rbase, RW)],
                        src_v.at[pl.ds(0, RW)])
        pltpu.sync_copy(e3_hbm.at[1, pl.ds(rbase, RW)],
                        dst_v.at[pl.ds(0, RW)])

        @pl.when(last)
        def _():
            pltpu.sync_copy(e3_hbm.at[0, pl.ds(rbase + RW, RX)],
                            src_v.at[pl.ds(RW, RX)])
            pltpu.sync_copy(e3_hbm.at[1, pl.ds(rbase + RW, RX)],
                            dst_v.at[pl.ds(RW, RX)])

        zeros = jnp.zeros((L,), jnp.float32)

        def zbody(i, _):
            r = i // (feat // L)
            k = i % (feat // L)
            zbuf_v[r, pl.ds(k * L, L)] = zeros
            return 0

        lax.fori_loop(0, (RPT // 5) * feat // L, zbody, 0)
        for p in range(5):
            pltpu.sync_copy(zbuf_v,
                            acc_sh.at[pl.ds(s * RPT + p * (RPT // 5), RPT // 5)])
        pltpu.sync_copy(g_hbm.at[pl.ds(s * RPT, RPT)],
                        gsrc_sh.at[pl.ds(s * RPT, RPT)])
        plsc.subcore_barrier()

        def issue_g(g, half, sem):
            for b in range(NB):
                pltpu.async_copy(gsrc_sh.at[src_v.at[g * NB + b]],
                                 rows_v.at[half * NB + b], sem)

        def drain_g(half, sem):
            for b in range(NB):
                pltpu.make_async_copy(gsrc_sh.at[src_v.at[0]],
                                      rows_v.at[half * NB + b], sem).wait()

        def issue_s(g, half, sem):
            for b in range(NB):
                pltpu.async_copy(rows_v.at[half * NB + b],
                                 acc_sh.at[dst_v.at[g * NB + b]], sem,
                                 add=True)

        def drain_s(half, sem):
            for b in range(NB):
                pltpu.make_async_copy(rows_v.at[half * NB + b],
                                      acc_sh.at[dst_v.at[0]], sem).wait()

        npair = jnp.where(last, RPAIR + 1, RPAIR)
        issue_g(0, 0, gsa)

        def body(k, _):
            g0 = 2 * k
            g1 = g0 + 1
            issue_g(g1, 1, gsb)
            drain_g(0, gsa)
            issue_s(g0, 0, ssa)

            @pl.when(k > 0)
            def _():
                drain_s(1, ssb)

            drain_g(1, gsb)
            issue_s(g1, 1, ssb)
            drain_s(0, ssa)

            @pl.when(k < npair - 1)
            def _():
                issue_g(g0 + 2, 0, gsa)

            return 0

        lax.fori_loop(0, npair, body, 0)
        drain_s(1, ssb)

        tb = npair * 2 * NB
        for b in range(NB):
            pltpu.async_copy(gsrc_sh.at[src_v.at[tb + b]], rows_v.at[b], gsa)
        drain_g(0, gsa)
        for b in range(NB):
            pltpu.sync_copy(rows_v.at[b], acc_sh.at[dst_v.at[tb + b]],
                            add=True)
        plsc.subcore_barrier()

        pltpu.sync_copy(acc_sh.at[pl.ds(s * RPT, RPT)],
                        out_hbm.at[c, pl.ds(s * RPT, RPT)])

    return _msg_kernel


_msg32 = _make_msg_kernel(H)
_msg16 = _make_msg_kernel(CPAD)


def _lin1_body(hist_ref, x_ref, w1_ref, g1_ref, dinv_ref):
    deg = jnp.sum(hist_ref[...], axis=0) + 1.0
    dinv = lax.rsqrt(deg)
    g1 = jnp.dot(x_ref[...], w1_ref[...],
                 preferred_element_type=jnp.float32) * dinv[:, None]
    g1_ref[...] = g1
    dinv_ref[...] = jnp.broadcast_to(dinv[:, None], (N, 8))


def _lin1(hist, x, W1):
    return pl.pallas_call(
        _lin1_body,
        out_shape=[
            jax.ShapeDtypeStruct((N, H), jnp.float32),
            jax.ShapeDtypeStruct((N, 8), jnp.float32),
        ],
    )(hist, x, W1)


BN = 1000


def _lin2_body(acc_ref, g1_ref, dinv_ref, b1_ref, w2_ref, g2_ref):
    dinv = dinv_ref[:, :1]
    a = acc_ref[0] + acc_ref[1] + g1_ref[...]
    h = jnp.maximum(a * dinv + b1_ref[...], 0.0)
    g2_ref[...] = jnp.dot(h, w2_ref[...],
                          preferred_element_type=jnp.float32) * dinv


def _lin2(acc, g1, dinv, b1_2d, W2p):
    return pl.pallas_call(
        _lin2_body,
        grid=(N // BN,),
        in_specs=[
            pl.BlockSpec((NC, BN, H), lambda i: (0, i, 0)),
            pl.BlockSpec((BN, H), lambda i: (i, 0)),
            pl.BlockSpec((BN, 8), lambda i: (i, 0)),
            pl.BlockSpec((1, H), lambda i: (0, 0)),
            pl.BlockSpec((H, CPAD), lambda i: (0, 0)),
        ],
        out_specs=pl.BlockSpec((BN, CPAD), lambda i: (i, 0)),
        out_shape=jax.ShapeDtypeStruct((N, CPAD), jnp.float32),
    )(acc, g1, dinv, b1_2d, W2p)


def _out_body(acc_ref, g2_ref, dinv_ref, b2_ref, out_ref):
    dinv = dinv_ref[:, :1]
    t = (acc_ref[0] + acc_ref[1] + g2_ref[...]) * dinv + b2_ref[...]
    mask = lax.broadcasted_iota(jnp.int32, (BN, CPAD), 1) < C
    neg = jnp.float32(-1e30)
    mx = jnp.max(jnp.where(mask, t, neg), axis=1, keepdims=True)
    e = jnp.where(mask, jnp.exp(t - mx), 0.0)
    lse = jnp.log(jnp.sum(e, axis=1, keepdims=True))
    out_ref[...] = (t - mx - lse)[:, :C]


def _outstage(acc, g2, dinv, b2p):
    return pl.pallas_call(
        _out_body,
        grid=(N // BN,),
        in_specs=[
            pl.BlockSpec((NC, BN, CPAD), lambda i: (0, i, 0)),
            pl.BlockSpec((BN, CPAD), lambda i: (i, 0)),
            pl.BlockSpec((BN, 8), lambda i: (i, 0)),
            pl.BlockSpec((1, CPAD), lambda i: (0, 0)),
        ],
        out_specs=pl.BlockSpec((BN, C), lambda i: (i, 0)),
        out_shape=jax.ShapeDtypeStruct((N, C), jnp.float32),
    )(acc, g2, dinv, b2p)


@jax.jit
def kernel(x, edge_index, W1, b1, W2, b2):
    e3 = edge_index.reshape(2, NROWS, CH)

    b1_2d = b1.reshape(1, H)
    W2p = jnp.zeros((H, CPAD), jnp.float32).at[:, :C].set(W2)
    b2p = jnp.zeros((1, CPAD), jnp.float32).at[:, :C].set(b2)

    hist = _deg_kernel(e3)
    g1, dinv = _lin1(hist, x, W1)
    acc1 = _msg32(g1, e3)
    g2 = _lin2(acc1, g1, dinv, b1_2d, W2p)
    acc2 = _msg16(g2, e3)
    return _outstage(acc2, g2, dinv, b2p)


# speedup vs baseline: 1.0668x; 1.0668x over previous
import functools

import jax
import jax.numpy as jnp
from jax import lax
from jax.experimental import pallas as pl
from jax.experimental.pallas import tpu as pltpu
from jax.experimental.pallas import tpu_sc as plsc

N = 10000
E = 640000
F_IN = 128
H = 32
C = 7
CPAD = 16

NC = 2
NS = 16
NW = NC * NS
L = 16

CH = 128
NROWS = E // CH
RW = 156
RX = NROWS - NW * RW
RMAX = RW + RX
NB = 4
RPAIR = RW // (2 * NB)
RPT = N // NS
NACC = N

_mesh = plsc.VectorSubcoreMesh(core_axis_name="c", subcore_axis_name="s")


@functools.partial(
    pl.kernel,
    out_type=jax.ShapeDtypeStruct((NW, N), jnp.float32),
    mesh=_mesh,
    compiler_params=pltpu.CompilerParams(needs_layout_passes=False, use_tc_tiling_on_sc=False),
    scratch_types=[
        pltpu.VMEM((RMAX, CH), jnp.int32),
        pltpu.VMEM((N,), jnp.float32),
    ],
)
def _deg_kernel(e3_hbm, out_hbm, dst_v, hist_v):
    wid = lax.axis_index("c") * NS + lax.axis_index("s")
    last = wid == NW - 1
    rbase = wid * RW
    pltpu.sync_copy(e3_hbm.at[1, pl.ds(rbase, RW)], dst_v.at[pl.ds(0, RW)])

    @pl.when(last)
    def _():
        pltpu.sync_copy(e3_hbm.at[1, pl.ds(rbase + RW, RX)],
                        dst_v.at[pl.ds(RW, RX)])

    zeros = jnp.zeros((L,), jnp.float32)

    def zbody(i, _):
        hist_v[pl.ds(i * L, L)] = zeros
        return 0

    lax.fori_loop(0, N // L, zbody, 0)

    ones = jnp.ones((L,), jnp.float32)
    nrows = jnp.where(last, RMAX, RW)

    def body(i, _):
        for k in range(CH // L):
            plsc.addupdate_scatter(hist_v, [dst_v[i, pl.ds(k * L, L)]], ones)
        return 0

    lax.fori_loop(0, nrows, body, 0)
    pltpu.sync_copy(hist_v, out_hbm.at[wid])


def _make_msg_kernel(feat):
    @functools.partial(
        pl.kernel,
        out_type=jax.ShapeDtypeStruct((NC, N, feat), jnp.float32),
        mesh=_mesh,
        compiler_params=pltpu.CompilerParams(needs_layout_passes=False, use_tc_tiling_on_sc=False),
        scratch_types=[
            pltpu.VMEM((RMAX, CH), jnp.int32),
            pltpu.VMEM((RMAX, CH), jnp.int32),
            pltpu.VMEM((2 * NB, CH, feat), jnp.float32),
            pltpu.VMEM((RPT // 5, feat), jnp.float32),
            pltpu.VMEM_SHARED((NACC, feat), jnp.float32),
            pltpu.VMEM_SHARED((N, feat), jnp.float32),
            pltpu.SemaphoreType.DMA,
            pltpu.SemaphoreType.DMA,
            pltpu.SemaphoreType.DMA,
            pltpu.SemaphoreType.DMA,
        ],
    )
    def _msg_kernel(g_hbm, e3_hbm, out_hbm,
                    src_v, dst_v, rows_v, zbuf_v, acc_sh, gsrc_sh,
                    gsa, gsb, ssa, ssb):
        c = lax.axis_index("c")
        s = lax.axis_index("s")
        wid = c * NS + s
        last = wid == NW - 1
        rbase = wid * RW

        pltpu.sync_copy(e3_hbm.at[0, pl.ds(rbase, RW)],
                        src_v.at[pl.ds(0, RW)])
        pltpu.sync_copy(e3_hbm.at[1, pl.ds(rbase, RW)],
                        dst_v.at[pl.ds(0, RW)])

        @pl.when(last)
        def _():
            pltpu.sync_copy(e3_hbm.at[0, pl.ds(rbase + RW, RX)],
                            src_v.at[pl.ds(RW, RX)])
            pltpu.sync_copy(e3_hbm.at[1, pl.ds(rbase + RW, RX)],
                            dst_v.at[pl.ds(RW, RX)])

        zeros = jnp.zeros((L,), jnp.float32)

        def zbody(i, _):
            r = i // (feat // L)
            k = i % (feat // L)
            zbuf_v[r, pl.ds(k * L, L)] = zeros
            return 0

        lax.fori_loop(0, (RPT // 5) * feat // L, zbody, 0)
        for p in range(5):
            pltpu.sync_copy(zbuf_v,
                            acc_sh.at[pl.ds(s * RPT + p * (RPT // 5), RPT // 5)])
        pltpu.sync_copy(g_hbm.at[pl.ds(s * RPT, RPT)],
                        gsrc_sh.at[pl.ds(s * RPT, RPT)])
        plsc.subcore_barrier()

        def issue_g(g, half, sem):
            for b in range(NB):
                pltpu.async_copy(gsrc_sh.at[src_v.at[g * NB + b]],
                                 rows_v.at[half * NB + b], sem)

        def drain_g(half, sem):
            for b in range(NB):
                pltpu.make_async_copy(gsrc_sh.at[src_v.at[0]],
                                      rows_v.at[half * NB + b], sem).wait()

        def issue_s(g, half, sem):
            for b in range(NB):
                pltpu.async_copy(rows_v.at[half * NB + b],
                                 acc_sh.at[dst_v.at[g * NB + b]], sem,
                                 add=True)

        def drain_s(half, sem):
            for b in range(NB):
                pltpu.make_async_copy(rows_v.at[half * NB + b],
                                      acc_sh.at[dst_v.at[0]], sem).wait()

        npair = jnp.where(last, RPAIR + 1, RPAIR)
        issue_g(0, 0, gsa)

        def body(k, _):
            g0 = 2 * k
            g1 = g0 + 1
            issue_g(g1, 1, gsb)
            drain_g(0, gsa)
            issue_s(g0, 0, ssa)

            @pl.when(k > 0)
            def _():
                drain_s(1, ssb)

            drain_g(1, gsb)
            issue_s(g1, 1, ssb)
            drain_s(0, ssa)

            @pl.when(k < npair - 1)
            def _():
                issue_g(g0 + 2, 0, gsa)

            return 0

        lax.fori_loop(0, npair, body, 0)
        drain_s(1, ssb)

        tb = npair * 2 * NB
        for b in range(NB):
            pltpu.async_copy(gsrc_sh.at[src_v.at[tb + b]], rows_v.at[b], gsa)
        drain_g(0, gsa)
        for b in range(NB):
            pltpu.sync_copy(rows_v.at[b], acc_sh.at[dst_v.at[tb + b]],
                            add=True)
        plsc.subcore_barrier()

        pltpu.sync_copy(acc_sh.at[pl.ds(s * RPT, RPT)],
                        out_hbm.at[c, pl.ds(s * RPT, RPT)])

    return _msg_kernel


_msg32 = _make_msg_kernel(H)
_msg16 = _make_msg_kernel(CPAD)


def _lin1_body(hist_ref, x_ref, w1_ref, g1_ref, dinv_ref):
    deg = jnp.sum(hist_ref[...], axis=0) + 1.0
    dinv = lax.rsqrt(deg)
    g1 = jnp.dot(x_ref[...], w1_ref[...],
                 preferred_element_type=jnp.float32) * dinv[:, None]
    g1_ref[...] = g1
    dinv_ref[...] = jnp.broadcast_to(dinv[:, None], (N, CPAD))


def _lin1(hist, x, W1):
    return pl.pallas_call(
        _lin1_body,
        out_shape=[
            jax.ShapeDtypeStruct((N, H), jnp.float32),
            jax.ShapeDtypeStruct((N, CPAD), jnp.float32),
        ],
    )(hist, x, W1)


BN = 1000


def _lin2_body(acc_ref, g1_ref, dinv_ref, b1_ref, w2_ref, g2_ref):
    dinv = dinv_ref[:, :1]
    a = acc_ref[0] + acc_ref[1] + g1_ref[...]
    h = jnp.maximum(a * dinv + b1_ref[...], 0.0)
    g2_ref[...] = jnp.dot(h, w2_ref[...],
                          preferred_element_type=jnp.float32) * dinv


def _lin2(acc, g1, dinv, b1_2d, W2p):
    return pl.pallas_call(
        _lin2_body,
        out_shape=jax.ShapeDtypeStruct((N, CPAD), jnp.float32),
    )(acc, g1, dinv, b1_2d, W2p)


NR = N * CPAD // CH


def _out_body(acc_ref, g2_ref, dinv_ref, b2_ref, gs_ref, out_ref):
    t = (acc_ref[0] + acc_ref[1] + g2_ref[...]) * dinv_ref[...] + b2_ref[...]
    lane = lax.broadcasted_iota(jnp.int32, (NR, CH), 1)
    mask = (lane % CPAD) < C
    neg = jnp.float32(-1e30)
    m = jnp.max(jnp.where(mask, t, neg), axis=1, keepdims=True)
    e = jnp.where(mask, jnp.exp(t - m), 0.0)
    s = jnp.dot(e, gs_ref[...], preferred_element_type=jnp.float32)
    out_ref[...] = t - m - jnp.log(s)


def _outstage(acc, g2, dinv, b2t, gsum):
    return pl.pallas_call(
        _out_body,
        out_shape=jax.ShapeDtypeStruct((NR, CH), jnp.float32),
    )(acc, g2, dinv, b2t, gsum)


@jax.jit
def kernel(x, edge_index, W1, b1, W2, b2):
    e3 = edge_index.reshape(2, NROWS, CH)

    b1_2d = b1.reshape(1, H)
    W2p = jnp.zeros((H, CPAD), jnp.float32).at[:, :C].set(W2)
    b2p = jnp.zeros((1, CPAD), jnp.float32).at[:, :C].set(b2)
    b2t = jnp.tile(b2p, (1, CH // CPAD))
    gid = jnp.arange(CH, dtype=jnp.int32) // CPAD
    gsum = (gid[:, None] == gid[None, :]).astype(jnp.float32)

    hist = _deg_kernel(e3)
    g1, dinv = _lin1(hist, x, W1)
    acc1 = _msg32(g1, e3)
    g2 = _lin2(acc1, g1, dinv, b1_2d, W2p)
    acc2 = _msg16(g2, e3)
    out128 = _outstage(acc2.reshape(NC, NR, CH), g2.reshape(NR, CH),
                       dinv.reshape(NR, CH), b2t, gsum)
    return out128.reshape(N, CPAD)[:, :C]
